# baseline (speedup 1.0000x reference)
import functools

import jax
import jax.numpy as jnp
from jax import lax
from jax.experimental import pallas as pl
from jax.experimental.pallas import tpu as pltpu
from jax.experimental.pallas import tpu_sc as plsc

B, S, D, N = 4, 8192, 1024, 50
NR = N - 1
WPB = 8
NW = B * WPB
RPW = 7
SLOTS = 8
CHUNKS = D // 16


def _sc_gather_dots(table_hbm, sep_hbm, out_hbm, sep_v, idx_v, rows_v,
                    res_v, sem):
    wid = lax.axis_index("s") * 2 + lax.axis_index("c")
    b = wid // WPB
    u = wid % WPB

    pltpu.sync_copy(sep_hbm, sep_v)
    lane = lax.iota(jnp.int32, 16)

    dnums = lax.GatherDimensionNumbers(
        offset_dims=(), collapsed_slice_dims=(0,), start_index_map=(0,))

    def lane_shuffle(x, perm):
        return lax.gather(
            x, perm[:, None], dnums, slice_sizes=(1,),
            mode=lax.GatherScatterMode.PROMISE_IN_BOUNDS,
            unique_indices=True)

    win = sep_v[b, pl.ds(u * RPW, 16)]
    win_d = sep_v[b, pl.ds(NR + 0 * u, 16)]
    dsel = lane_shuffle(win_d, jnp.zeros((16,), jnp.int32))
    limit = jnp.where(u < RPW, RPW, 0)
    vals = jnp.where(lane < limit, win, dsel)
    idx_v[...] = jnp.clip(vals + b * S, 0, B * S - 1)

    pltpu.async_copy(table_hbm.at[idx_v.at[pl.ds(0, SLOTS)]], rows_v,
                     sem).wait()

    zero = jnp.zeros((16,), jnp.float32)

    def acc_chunk(c, nums, rns):
        dchunk = rows_v[SLOTS - 1, pl.ds(c * 16, 16)]
        new_nums = []
        new_rns = []
        for r in range(SLOTS - 1):
            rchunk = rows_v[r, pl.ds(c * 16, 16)]
            new_nums.append(nums[r] + rchunk * dchunk)
            new_rns.append(rns[r] + rchunk * rchunk)
        new_nums.append(nums[SLOTS - 1] + dchunk * dchunk)
        return new_nums, new_rns

    def chunk_body(c, carry):
        nums, rns = carry
        for t in range(4):
            nums, rns = acc_chunk(4 * c + t, nums, rns)
        return tuple(nums), tuple(rns)

    nums0 = tuple(zero for _ in range(SLOTS))
    rns0 = tuple(zero for _ in range(SLOTS - 1))
    nums, rns = lax.fori_loop(0, CHUNKS // 4, chunk_body, (nums0, rns0))

    vecs = list(nums) + list(rns) + [zero]
    s = 1
    while len(vecs) > 1:
        nxt = []
        for i in range(0, len(vecs), 2):
            a, b2 = vecs[i], vecs[i + 1]
            ra = a + lane_shuffle(a, lane ^ s)
            rb = b2 + lane_shuffle(b2, lane ^ s)
            nxt.append(jnp.where((lane & s) == 0, ra, rb))
        vecs = nxt
        s *= 2
    res_v[...] = vecs[0]
    pltpu.sync_copy(res_v.at[pl.ds(0, 8)],
                    out_hbm.at[b, pl.ds(8 * u, 8)])
    pltpu.sync_copy(res_v.at[pl.ds(8, 8)],
                    out_hbm.at[b, pl.ds(64 + 8 * u, 8)])


_sc_call = functools.partial(
    pl.kernel,
    mesh=plsc.VectorSubcoreMesh(core_axis_name="c", subcore_axis_name="s"),
    out_type=jax.ShapeDtypeStruct((B, 128), jnp.float32),
    scratch_types=[
        pltpu.VMEM((B, N), jnp.int32),
        pltpu.VMEM((16,), jnp.int32),
        pltpu.VMEM((SLOTS, D), jnp.float32),
        pltpu.VMEM((16,), jnp.float32),
        pltpu.SemaphoreType.DMA,
    ],
)(_sc_gather_dots)


def _tc_finish(o_ref, lab_ref, sim_ref, loss_ref):
    o = o_ref[...]
    num = o[:, 0:64]
    rn2 = o[:, 64:128]
    dn2 = num[:, RPW:SLOTS]
    col = lax.broadcasted_iota(jnp.int32, (B, 64), 1)
    uu = lax.div(col, WPB)
    kk = lax.rem(col, WPB)
    valid = (uu < RPW) & (kk < RPW)
    raw = num / jnp.maximum(jnp.sqrt(rn2 * dn2), 1e-6)
    sim = jnp.where(valid, raw, 0.0)
    e = jnp.where(valid, jnp.exp(sim), 0.0)
    lse = jnp.log(jnp.sum(e, axis=1, keepdims=True))
    jpos = uu * RPW + kk
    row = lax.broadcasted_iota(jnp.int32, (B, 64), 0)
    labv = jnp.where(row == 0, lab_ref[0],
                     jnp.where(row == 1, lab_ref[1],
                               jnp.where(row == 2, lab_ref[2], lab_ref[3])))
    match = valid & (jpos == labv)
    picked = jnp.sum(jnp.where(match, sim, 0.0), axis=1,
                     keepdims=True)
    loss_ref[...] = jnp.sum(lse - picked, axis=0, keepdims=True) / B
    p_c = lax.broadcasted_iota(jnp.int32, (64, NR), 0)
    p_j = lax.broadcasted_iota(jnp.int32, (64, NR), 1)
    perm = (p_c == WPB * lax.div(p_j, RPW)
            + lax.rem(p_j, RPW)).astype(jnp.float32)
    sim_ref[...] = jax.lax.dot(sim, perm,
                               precision=jax.lax.Precision.HIGHEST,
                               preferred_element_type=jnp.float32)


def kernel(sequence_output, sep_positions, labels):
    table = sequence_output.reshape(B * S, D)
    out = _sc_call(table, sep_positions.astype(jnp.int32))
    sim_scores, loss = pl.pallas_call(
        _tc_finish,
        in_specs=[
            pl.BlockSpec(memory_space=pltpu.VMEM),
            pl.BlockSpec(memory_space=pltpu.SMEM),
        ],
        out_shape=[
            jax.ShapeDtypeStruct((B, NR), jnp.float32),
            jax.ShapeDtypeStruct((1, 1), jnp.float32),
        ],
    )(out, labels.astype(jnp.int32))
    return sim_scores, loss[0, 0]


# speedup vs baseline: 1.0009x; 1.0009x over previous
import functools

import jax
import jax.numpy as jnp
from jax import lax
from jax.experimental import pallas as pl
from jax.experimental.pallas import tpu as pltpu
from jax.experimental.pallas import tpu_sc as plsc

B, S, D, N = 4, 8192, 1024, 50
NR = N - 1
WPB = 8
NW = B * WPB
RPW = 7
SLOTS = 8
CHUNKS = D // 16


def _sc_gather_dots(table_hbm, sep_hbm, out_hbm, sep_v, idx_v, rows_v,
                    res_v, sem):
    wid = lax.axis_index("s") * 2 + lax.axis_index("c")
    b = wid // WPB
    u = wid % WPB

    pltpu.sync_copy(sep_hbm, sep_v)
    lane = lax.iota(jnp.int32, 16)

    dnums = lax.GatherDimensionNumbers(
        offset_dims=(), collapsed_slice_dims=(0,), start_index_map=(0,))

    def lane_shuffle(x, perm):
        return lax.gather(
            x, perm[:, None], dnums, slice_sizes=(1,),
            mode=lax.GatherScatterMode.PROMISE_IN_BOUNDS,
            unique_indices=True)

    win = sep_v[b, pl.ds(u * RPW, 16)]
    win_d = sep_v[b, pl.ds(NR + 0 * u, 16)]
    dsel = lane_shuffle(win_d, jnp.zeros((16,), jnp.int32))
    limit = jnp.where(u < RPW, RPW, 0)
    vals = jnp.where(lane < limit, win, dsel)
    idx_v[...] = jnp.clip(vals + b * S, 0, B * S - 1)

    pltpu.async_copy(table_hbm.at[idx_v.at[pl.ds(0, SLOTS)]], rows_v,
                     sem).wait()

    zero = jnp.zeros((16,), jnp.float32)

    def acc_chunk(c, nums, rns):
        dchunk = rows_v[SLOTS - 1, pl.ds(c * 16, 16)]
        new_nums = []
        new_rns = []
        for r in range(SLOTS - 1):
            rchunk = rows_v[r, pl.ds(c * 16, 16)]
            new_nums.append(nums[r] + rchunk * dchunk)
            new_rns.append(rns[r] + rchunk * rchunk)
        new_nums.append(nums[SLOTS - 1] + dchunk * dchunk)
        return new_nums, new_rns

    def chunk_body(c, carry):
        nums, rns = carry
        nums, rns = acc_chunk(2 * c, nums, rns)
        nums, rns = acc_chunk(2 * c + 1, nums, rns)
        return tuple(nums), tuple(rns)

    nums0 = tuple(zero for _ in range(SLOTS))
    rns0 = tuple(zero for _ in range(SLOTS - 1))
    nums, rns = lax.fori_loop(0, CHUNKS // 2, chunk_body, (nums0, rns0))

    vecs = list(nums) + list(rns) + [zero]
    s = 1
    while len(vecs) > 1:
        nxt = []
        for i in range(0, len(vecs), 2):
            a, b2 = vecs[i], vecs[i + 1]
            ra = a + lane_shuffle(a, lane ^ s)
            rb = b2 + lane_shuffle(b2, lane ^ s)
            nxt.append(jnp.where((lane & s) == 0, ra, rb))
        vecs = nxt
        s *= 2
    res_v[...] = vecs[0]
    pltpu.sync_copy(res_v.at[pl.ds(0, 8)],
                    out_hbm.at[b, pl.ds(8 * u, 8)])
    pltpu.sync_copy(res_v.at[pl.ds(8, 8)],
                    out_hbm.at[b, pl.ds(64 + 8 * u, 8)])


_sc_call = functools.partial(
    pl.kernel,
    mesh=plsc.VectorSubcoreMesh(core_axis_name="c", subcore_axis_name="s"),
    out_type=jax.ShapeDtypeStruct((B, 128), jnp.float32),
    scratch_types=[
        pltpu.VMEM((B, N), jnp.int32),
        pltpu.VMEM((16,), jnp.int32),
        pltpu.VMEM((SLOTS, D), jnp.float32),
        pltpu.VMEM((16,), jnp.float32),
        pltpu.SemaphoreType.DMA,
    ],
)(_sc_gather_dots)


def _tc_finish(o_ref, lab_ref, sim_ref, loss_ref):
    o = o_ref[...]
    num = o[:, 0:64]
    rn2 = o[:, 64:128]
    dn2 = num[:, RPW:SLOTS]
    col = lax.broadcasted_iota(jnp.int32, (B, 64), 1)
    uu = lax.div(col, WPB)
    kk = lax.rem(col, WPB)
    valid = (uu < RPW) & (kk < RPW)
    raw = num / jnp.maximum(jnp.sqrt(rn2 * dn2), 1e-6)
    sim = jnp.where(valid, raw, 0.0)
    e = jnp.where(valid, jnp.exp(sim), 0.0)
    lse = jnp.log(jnp.sum(e, axis=1, keepdims=True))
    jpos = uu * RPW + kk
    match = valid & (jpos == lab_ref[...])
    picked = jnp.sum(jnp.where(match, sim, 0.0), axis=1,
                     keepdims=True)
    loss_ref[...] = jnp.sum(lse - picked, axis=0, keepdims=True) / B
    p_c = lax.broadcasted_iota(jnp.int32, (64, NR), 0)
    p_j = lax.broadcasted_iota(jnp.int32, (64, NR), 1)
    perm = (p_c == WPB * lax.div(p_j, RPW)
            + lax.rem(p_j, RPW)).astype(jnp.float32)
    sim_ref[...] = jax.lax.dot(sim, perm,
                               precision=jax.lax.Precision.HIGHEST,
                               preferred_element_type=jnp.float32)


def kernel(sequence_output, sep_positions, labels):
    table = sequence_output.reshape(B * S, D)
    out = _sc_call(table, sep_positions.astype(jnp.int32))
    sim_scores, loss = pl.pallas_call(
        _tc_finish,
        out_shape=[
            jax.ShapeDtypeStruct((B, NR), jnp.float32),
            jax.ShapeDtypeStruct((1, 1), jnp.float32),
        ],
    )(out, labels.astype(jnp.int32)[:, None])
    return sim_scores, loss[0, 0]
